# baseline (speedup 1.0000x reference)
import jax
import jax.numpy as jnp
from jax import lax
from jax.experimental import pallas as pl
from jax.experimental.pallas import tpu as pltpu
from jax.experimental.pallas import tpu_sc as plsc

T = 12
N = 10000
E = 320000
NP = 10240
H = 128
H2 = 256
B = 1024
NB = NP // B
NC = 2
NS = 16
CH = 80
E2 = 327680
NCH = 128
ER = E2 // CH
ATOT = 10112
WR = ATOT // NS
AC0 = 232
AC1 = 24


def _deg_body(dstr, ones80, zeros128, out, dst_v, ones_v, sem, acc):
    c = lax.axis_index("c")
    s = lax.axis_index("s")
    w = c * NS + s
    pltpu.sync_copy(ones80, ones_v)

    def per_t(t, carry):
        pltpu.sync_copy(zeros128, acc.at[pl.ds(s * WR, WR), :])
        plsc.subcore_barrier()
        base_row = t * ER + w * NCH
        pltpu.sync_copy(dstr.at[pl.ds(base_row, NCH)], dst_v)

        def group(g, carry):
            for u in range(4):
                i = g * 4 + u
                pltpu.make_async_copy(
                    ones_v, acc.at[dst_v.at[i]], sem).start(add=True)
            for _u in range(4):
                pltpu.make_async_copy(
                    ones_v, acc.at[dst_v.at[0]], sem).wait()
            return carry
        lax.fori_loop(0, NCH // 4, group, 0)
        plsc.subcore_barrier()
        ob = (c * T + t) * NP + s * WR
        pltpu.sync_copy(acc.at[pl.ds(s * WR, WR), :],
                        out.at[pl.ds(ob, WR), :])
        return carry
    lax.fori_loop(0, T, per_t, 0)


def _emb_body(catp, num2, f0, f1, f2, xc012, x3,
              i0v, i1v, i2v, buf, nv, sm0, sm1, sm2, smn):
    c = lax.axis_index("c")
    s = lax.axis_index("s")
    w = c * NS + s

    def unit(k, carry):
        uid = w + NC * NS * k
        t = uid // 16
        rb = uid - t * 16
        pltpu.sync_copy(catp.at[pl.ds((t * 3 + 0) * 128 + rb * 8, 8)], i0v)
        pltpu.sync_copy(catp.at[pl.ds((t * 3 + 1) * 128 + rb * 8, 8)], i1v)
        pltpu.sync_copy(catp.at[pl.ds((t * 3 + 2) * 128 + rb * 8, 8)], i2v)
        for u in range(8):
            r = rb * 8 + u

            @pl.when(r < 125)
            def _(r=r, u=u):
                d0 = pltpu.make_async_copy(f0.at[i0v.at[u]], buf, sm0)
                d0.start()
                dn = pltpu.make_async_copy(
                    num2.at[pl.ds(t * N + r * CH, CH), :], nv, smn)
                dn.start()
                d0.wait()
                d1 = pltpu.make_async_copy(f1.at[i1v.at[u]], buf, sm1)
                d2 = pltpu.make_async_copy(f2.at[i2v.at[u]], buf, sm2)
                d1.start(add=True)
                d2.start(add=True)
                d1.wait()
                d2.wait()
                dn.wait()
                ob = t * NP + r * CH
                pltpu.sync_copy(buf, xc012.at[pl.ds(ob, CH), :])
                pltpu.sync_copy(nv, x3.at[pl.ds(ob, CH), :])
        return carry
    lax.fori_loop(0, T * 16 // (NC * NS), unit, 0)


def _agg_body(y, srcf, dstf, zeros128, out,
              si0, si1, si2, si3, di0, di1, di2, di3, r0, r1, r2, r3,
              i0, i1, i2, i3, g0, g1, g2, g3, s0, s1, s2, s3, acc):
    c = lax.axis_index("c")
    s = lax.axis_index("s")
    w = c * NS + s
    sis = (si0, si1, si2, si3)
    dis = (di0, di1, di2, di3)
    rbufs = (r0, r1, r2, r3)
    isems = (i0, i1, i2, i3)
    gsems = (g0, g1, g2, g3)
    ssems = (s0, s1, s2, s3)

    def idx_start(base, i, b):
        pltpu.make_async_copy(
            srcf.at[pl.ds(base + i * CH, CH)], sis[b], isems[b]).start()
        pltpu.make_async_copy(
            dstf.at[pl.ds(base + i * CH, CH)], dis[b], isems[b]).start()

    def idx_wait(b):
        pltpu.make_async_copy(srcf.at[pl.ds(0, CH)], sis[b], isems[b]).wait()
        pltpu.make_async_copy(dstf.at[pl.ds(0, CH)], dis[b], isems[b]).wait()

    def gather_start(b):
        pltpu.make_async_copy(y.at[sis[b]], rbufs[b], gsems[b]).start()

    def gather_wait(b):
        pltpu.make_async_copy(y.at[sis[b]], rbufs[b], gsems[b]).wait()

    def scat_start(b):
        pltpu.make_async_copy(
            rbufs[b], acc.at[dis[b]], ssems[b]).start(add=True)

    def scat_wait(b):
        pltpu.make_async_copy(
            rbufs[b], acc.at[dis[b]], ssems[b]).wait()

    ncht = jnp.where(c == 0, AC0, AC1)

    def per_t(t, carry):
        pltpu.sync_copy(zeros128, acc.at[pl.ds(s * WR, WR), :])
        plsc.subcore_barrier()
        base = t * E2 + jnp.where(
            c == 0, s * (AC0 * CH), NS * (AC0 * CH) + s * (AC1 * CH))
        idx_start(base, 0, 0)
        idx_start(base, 1, 1)
        idx_wait(0)
        gather_start(0)

        def grp(g4, carry):
            for u in range(4):
                i = g4 * 4 + u
                b = u
                b1 = (u + 1) % 4
                b2 = (u + 2) % 4

                @pl.when(i < ncht)
                def _(b=b):
                    gather_wait(b)
                    scat_start(b)

                @pl.when(jnp.logical_and(i - 2 >= 0, i - 2 < ncht))
                def _(b2=b2):
                    scat_wait(b2)

                @pl.when(i + 2 < ncht)
                def _(i=i, b2=b2):
                    idx_start(base, i + 2, b2)

                @pl.when(i + 1 < ncht)
                def _(b1=b1):
                    idx_wait(b1)
                    gather_start(b1)
            return carry
        lax.fori_loop(0, AC0 // 4 + 1, grp, 0)
        plsc.subcore_barrier()
        ob = (c * T + t) * NP + s * WR
        pltpu.sync_copy(acc.at[pl.ds(s * WR, WR), :],
                        out.at[pl.ds(ob, WR), :])
        return carry
    lax.fori_loop(0, T, per_t, 0)


def _prep_body(p0, p1, c012, x3, dinv_ref, y1_ref):
    deg = p0[:, 0:1] + p1[:, 0:1] + 1.0
    dinv = lax.rsqrt(deg)
    dinv_ref[...] = jnp.broadcast_to(dinv, (B, 8))
    xc = jnp.concatenate([c012[:, 0:96], x3[...]], axis=1)
    y1_ref[...] = dinv * xc


def _mid_body(p0, p1, y1, dv, w1, b1r, w2, y2_ref):
    dinv = dv[:, 0:1]
    s1 = dinv * (p0[...] + p1[...] + y1[...])
    g1 = jnp.maximum(
        jnp.dot(s1, w1[0], preferred_element_type=jnp.float32) + b1r[0], 0.0)
    h2 = jnp.dot(g1, w2[0], preferred_element_type=jnp.float32)
    y2_ref[...] = dinv * h2


def _gru_body(q0, q1, y2, dv, b2r, wih, whh, bihr, bhhr, out_ref, h_scr):
    t = pl.program_id(0)
    nb = pl.program_id(1)

    @pl.when(t == 0)
    def _():
        h_scr[pl.ds(nb * B, B), :] = jnp.zeros((B, H), jnp.float32)

    dinv = dv[:, 0:1]
    x_t = dinv * (q0[...] + q1[...] + y2[...]) + b2r[0]
    h = h_scr[pl.ds(nb * B, B), :]
    gi = jnp.dot(x_t, wih[...], preferred_element_type=jnp.float32) + bihr[...]
    gh = jnp.dot(h, whh[...], preferred_element_type=jnp.float32) + bhhr[...]
    r = jax.nn.sigmoid(gi[:, 0:H] + gh[:, 0:H])
    z = jax.nn.sigmoid(gi[:, H:2 * H] + gh[:, H:2 * H])
    n_ = jnp.tanh(gi[:, 2 * H:3 * H] + r * gh[:, 2 * H:3 * H])
    hn = (1.0 - z) * n_ + z * h
    h_scr[pl.ds(nb * B, B), :] = hn
    out_ref[...] = hn


def kernel(cat_x, num_x, edges, emb0, emb1, emb2,
           W1, b1, W2, b2, Wih, Whh, bih, bhh):
    f32 = jnp.float32
    i32 = jnp.int32
    cat_t = jnp.transpose(cat_x.astype(i32), (0, 2, 1))
    catp = jnp.pad(cat_t.reshape(T * 3, 125, CH),
                   ((0, 0), (0, 3), (0, 0))).reshape(T * 3 * 128, CH)
    num2 = num_x.reshape(T * N, 32)
    src = edges[:, 0, :].astype(i32)
    dst = edges[:, 1, :].astype(i32)
    epad = E2 - E
    srcp = jnp.concatenate([src, jnp.zeros((T, epad), i32)], axis=1)
    dstp = jnp.concatenate([dst, jnp.full((T, epad), N, i32)], axis=1)
    src_adj = srcp + (jnp.arange(T, dtype=i32) * NP)[:, None]
    srcf = src_adj.reshape(T * E2)
    dstf = dstp.reshape(T * E2)
    dstr = dstp.reshape(T * ER, CH)
    ones80 = jnp.ones((CH, H), f32)
    zeros128 = jnp.zeros((WR, H), f32)
    f0 = jnp.pad(emb0, ((0, 0), (0, 96)))
    f1 = jnp.pad(emb1, ((0, 0), (32, 64)))
    f2 = jnp.pad(emb2, ((0, 0), (64, 32)))

    mesh = plsc.VectorSubcoreMesh(core_axis_name="c", subcore_axis_name="s")

    degp = pl.kernel(
        _deg_body,
        out_type=jax.ShapeDtypeStruct((NC * T * NP, H), f32),
        mesh=mesh,
        scratch_types=[
            pltpu.VMEM((NCH, CH), i32),
            pltpu.VMEM((CH, H), f32),
            pltpu.SemaphoreType.DMA,
            pltpu.VMEM_SHARED((ATOT, H), f32),
        ],
    )(dstr, ones80, zeros128)

    xc012, x3 = pl.kernel(
        _emb_body,
        out_type=(jax.ShapeDtypeStruct((T * NP, H), f32),
                  jax.ShapeDtypeStruct((T * NP, 32), f32)),
        mesh=mesh,
        scratch_types=[
            pltpu.VMEM((8, CH), i32),
            pltpu.VMEM((8, CH), i32),
            pltpu.VMEM((8, CH), i32),
            pltpu.VMEM((CH, H), f32),
            pltpu.VMEM((CH, 32), f32),
            pltpu.SemaphoreType.DMA,
            pltpu.SemaphoreType.DMA,
            pltpu.SemaphoreType.DMA,
            pltpu.SemaphoreType.DMA,
        ],
    )(catp, num2, f0, f1, f2)

    dinvT, y1 = pl.pallas_call(
        _prep_body,
        grid=(T, NB),
        in_specs=[
            pl.BlockSpec((B, H), lambda t, nb: (t * NB + nb, 0)),
            pl.BlockSpec((B, H), lambda t, nb: ((T + t) * NB + nb, 0)),
            pl.BlockSpec((B, H), lambda t, nb: (t * NB + nb, 0)),
            pl.BlockSpec((B, 32), lambda t, nb: (t * NB + nb, 0)),
        ],
        out_specs=[
            pl.BlockSpec((B, 8), lambda t, nb: (t * NB + nb, 0)),
            pl.BlockSpec((B, H), lambda t, nb: (t * NB + nb, 0)),
        ],
        out_shape=[
            jax.ShapeDtypeStruct((T * NP, 8), f32),
            jax.ShapeDtypeStruct((T * NP, H), f32),
        ],
    )(degp, degp, xc012, x3)

    def agg(yarr):
        return pl.kernel(
            _agg_body,
            out_type=jax.ShapeDtypeStruct((NC * T * NP, H), f32),
            mesh=mesh,
            scratch_types=(
                [pltpu.VMEM((CH,), i32) for _ in range(8)]
                + [pltpu.VMEM((CH, H), f32) for _ in range(4)]
                + [pltpu.SemaphoreType.DMA for _ in range(12)]
                + [pltpu.VMEM_SHARED((ATOT, H), f32)]
            ),
        )(yarr, srcf, dstf, zeros128)

    P = agg(y1)

    y2 = pl.pallas_call(
        _mid_body,
        grid=(T, NB),
        in_specs=[
            pl.BlockSpec((B, H), lambda t, nb: (t * NB + nb, 0)),
            pl.BlockSpec((B, H), lambda t, nb: ((T + t) * NB + nb, 0)),
            pl.BlockSpec((B, H), lambda t, nb: (t * NB + nb, 0)),
            pl.BlockSpec((B, 8), lambda t, nb: (t * NB + nb, 0)),
            pl.BlockSpec((1, H, H2), lambda t, nb: (t, 0, 0)),
            pl.BlockSpec((1, 1, H2), lambda t, nb: (t, 0, 0)),
            pl.BlockSpec((1, H2, H), lambda t, nb: (t, 0, 0)),
        ],
        out_specs=pl.BlockSpec((B, H), lambda t, nb: (t * NB + nb, 0)),
        out_shape=jax.ShapeDtypeStruct((T * NP, H), f32),
    )(P, P, y1, dinvT, W1, b1.reshape(T, 1, H2), W2)

    Q = agg(y2)

    h = pl.pallas_call(
        _gru_body,
        grid=(T, NB),
        in_specs=[
            pl.BlockSpec((B, H), lambda t, nb: (t * NB + nb, 0)),
            pl.BlockSpec((B, H), lambda t, nb: ((T + t) * NB + nb, 0)),
            pl.BlockSpec((B, H), lambda t, nb: (t * NB + nb, 0)),
            pl.BlockSpec((B, 8), lambda t, nb: (t * NB + nb, 0)),
            pl.BlockSpec((1, 1, H), lambda t, nb: (t, 0, 0)),
            pl.BlockSpec((H, 3 * H), lambda t, nb: (0, 0)),
            pl.BlockSpec((H, 3 * H), lambda t, nb: (0, 0)),
            pl.BlockSpec((1, 3 * H), lambda t, nb: (0, 0)),
            pl.BlockSpec((1, 3 * H), lambda t, nb: (0, 0)),
        ],
        out_specs=pl.BlockSpec((B, H), lambda t, nb: (nb, 0)),
        out_shape=jax.ShapeDtypeStruct((NP, H), f32),
        scratch_shapes=[pltpu.VMEM((NP, H), f32)],
    )(Q, Q, y2, dinvT, b2.reshape(T, 1, H), jnp.transpose(Wih),
      jnp.transpose(Whh), bih.reshape(1, 3 * H), bhh.reshape(1, 3 * H))
    return h[:N]


# speedup vs baseline: 1.0077x; 1.0077x over previous
import jax
import jax.numpy as jnp
from jax import lax
from jax.experimental import pallas as pl
from jax.experimental.pallas import tpu as pltpu
from jax.experimental.pallas import tpu_sc as plsc

T = 12
N = 10000
E = 320000
NP = 10240
H = 128
H2 = 256
B = 1024
NB = NP // B
NC = 2
NS = 16
CH = 80
E2 = 327680
NCH = 128
ER = E2 // CH
ATOT = 10112
WR = ATOT // NS
AC0 = 228
AC1 = 28


def _deg_body(dstr, ones80, zeros128, out, dst_v, ones_v, sem, acc):
    c = lax.axis_index("c")
    s = lax.axis_index("s")
    w = c * NS + s
    pltpu.sync_copy(ones80, ones_v)

    def per_t(t, carry):
        pltpu.sync_copy(zeros128, acc.at[pl.ds(s * WR, WR), :])
        plsc.subcore_barrier()
        base_row = t * ER + w * NCH
        pltpu.sync_copy(dstr.at[pl.ds(base_row, NCH)], dst_v)

        def group(g, carry):
            for u in range(8):
                i = g * 8 + u
                pltpu.make_async_copy(
                    ones_v, acc.at[dst_v.at[i]], sem).start(add=True)
            for _u in range(8):
                pltpu.make_async_copy(
                    ones_v, acc.at[dst_v.at[0]], sem).wait()
            return carry
        lax.fori_loop(0, NCH // 8, group, 0)
        plsc.subcore_barrier()
        ob = (c * T + t) * NP + s * WR
        pltpu.sync_copy(acc.at[pl.ds(s * WR, WR), :],
                        out.at[pl.ds(ob, WR), :])
        return carry
    lax.fori_loop(0, T, per_t, 0)


def _emb_body(catp, num2, f0, f1, f2, xc012, x3,
              i0v, i1v, i2v, buf, nv, sm0, sm1, sm2, smn):
    c = lax.axis_index("c")
    s = lax.axis_index("s")
    w = c * NS + s

    def unit(k, carry):
        uid = w + NC * NS * k
        t = uid // 16
        rb = uid - t * 16
        pltpu.sync_copy(catp.at[pl.ds((t * 3 + 0) * 128 + rb * 8, 8)], i0v)
        pltpu.sync_copy(catp.at[pl.ds((t * 3 + 1) * 128 + rb * 8, 8)], i1v)
        pltpu.sync_copy(catp.at[pl.ds((t * 3 + 2) * 128 + rb * 8, 8)], i2v)
        for u in range(8):
            r = rb * 8 + u

            @pl.when(r < 125)
            def _(r=r, u=u):
                d0 = pltpu.make_async_copy(f0.at[i0v.at[u]], buf, sm0)
                d0.start()
                dn = pltpu.make_async_copy(
                    num2.at[pl.ds(t * N + r * CH, CH), :], nv, smn)
                dn.start()
                d0.wait()
                d1 = pltpu.make_async_copy(f1.at[i1v.at[u]], buf, sm1)
                d2 = pltpu.make_async_copy(f2.at[i2v.at[u]], buf, sm2)
                d1.start(add=True)
                d2.start(add=True)
                d1.wait()
                d2.wait()
                dn.wait()
                ob = t * NP + r * CH
                pltpu.sync_copy(buf, xc012.at[pl.ds(ob, CH), :])
                pltpu.sync_copy(nv, x3.at[pl.ds(ob, CH), :])
        return carry
    lax.fori_loop(0, T * 16 // (NC * NS), unit, 0)


def _agg_body(y, srcf, dstf, zeros128, out,
              si0, si1, si2, si3, di0, di1, di2, di3, r0, r1, r2, r3,
              i0, i1, i2, i3, g0, g1, g2, g3, s0, s1, s2, s3, acc):
    c = lax.axis_index("c")
    s = lax.axis_index("s")
    w = c * NS + s
    sis = (si0, si1, si2, si3)
    dis = (di0, di1, di2, di3)
    rbufs = (r0, r1, r2, r3)
    isems = (i0, i1, i2, i3)
    gsems = (g0, g1, g2, g3)
    ssems = (s0, s1, s2, s3)

    def idx_start(base, i, b):
        pltpu.make_async_copy(
            srcf.at[pl.ds(base + i * CH, CH)], sis[b], isems[b]).start()
        pltpu.make_async_copy(
            dstf.at[pl.ds(base + i * CH, CH)], dis[b], isems[b]).start()

    def idx_wait(b):
        pltpu.make_async_copy(srcf.at[pl.ds(0, CH)], sis[b], isems[b]).wait()
        pltpu.make_async_copy(dstf.at[pl.ds(0, CH)], dis[b], isems[b]).wait()

    def gather_start(b):
        pltpu.make_async_copy(y.at[sis[b]], rbufs[b], gsems[b]).start()

    def gather_wait(b):
        pltpu.make_async_copy(y.at[sis[b]], rbufs[b], gsems[b]).wait()

    def scat_start(b):
        pltpu.make_async_copy(
            rbufs[b], acc.at[dis[b]], ssems[b]).start(add=True)

    def scat_wait(b):
        pltpu.make_async_copy(
            rbufs[b], acc.at[dis[b]], ssems[b]).wait()

    ncht = jnp.where(c == 0, AC0, AC1)

    def per_t(t, carry):
        pltpu.sync_copy(zeros128, acc.at[pl.ds(s * WR, WR), :])
        plsc.subcore_barrier()
        base = t * E2 + jnp.where(
            c == 0, s * (AC0 * CH), NS * (AC0 * CH) + s * (AC1 * CH))
        idx_start(base, 0, 0)
        idx_start(base, 1, 1)
        idx_wait(0)
        gather_start(0)

        def grp(g4, carry):
            for u in range(4):
                i = g4 * 4 + u
                b = u
                b1 = (u + 1) % 4
                b2 = (u + 2) % 4

                @pl.when(i < ncht)
                def _(b=b):
                    gather_wait(b)
                    scat_start(b)

                @pl.when(jnp.logical_and(i - 2 >= 0, i - 2 < ncht))
                def _(b2=b2):
                    scat_wait(b2)

                @pl.when(i + 2 < ncht)
                def _(i=i, b2=b2):
                    idx_start(base, i + 2, b2)

                @pl.when(i + 1 < ncht)
                def _(b1=b1):
                    idx_wait(b1)
                    gather_start(b1)
            return carry
        lax.fori_loop(0, AC0 // 4 + 1, grp, 0)
        plsc.subcore_barrier()
        ob = (c * T + t) * NP + s * WR
        pltpu.sync_copy(acc.at[pl.ds(s * WR, WR), :],
                        out.at[pl.ds(ob, WR), :])
        return carry
    lax.fori_loop(0, T, per_t, 0)


def _prep_body(p0, p1, c012, x3, dinv_ref, y1_ref):
    deg = p0[:, 0:1] + p1[:, 0:1] + 1.0
    dinv = lax.rsqrt(deg)
    dinv_ref[...] = jnp.broadcast_to(dinv, (B, 8))
    xc = jnp.concatenate([c012[:, 0:96], x3[...]], axis=1)
    y1_ref[...] = dinv * xc


def _mid_body(p0, p1, y1, dv, w1, b1r, w2, y2_ref):
    dinv = dv[:, 0:1]
    s1 = dinv * (p0[...] + p1[...] + y1[...])
    g1 = jnp.maximum(
        jnp.dot(s1, w1[0], preferred_element_type=jnp.float32) + b1r[0], 0.0)
    h2 = jnp.dot(g1, w2[0], preferred_element_type=jnp.float32)
    y2_ref[...] = dinv * h2


def _gru_body(q0, q1, y2, dv, b2r, wih, whh, bihr, bhhr, out_ref, h_scr):
    t = pl.program_id(0)
    nb = pl.program_id(1)

    @pl.when(t == 0)
    def _():
        h_scr[pl.ds(nb * B, B), :] = jnp.zeros((B, H), jnp.float32)

    dinv = dv[:, 0:1]
    x_t = dinv * (q0[...] + q1[...] + y2[...]) + b2r[0]
    h = h_scr[pl.ds(nb * B, B), :]
    gi = jnp.dot(x_t, wih[...], preferred_element_type=jnp.float32) + bihr[...]
    gh = jnp.dot(h, whh[...], preferred_element_type=jnp.float32) + bhhr[...]
    r = jax.nn.sigmoid(gi[:, 0:H] + gh[:, 0:H])
    z = jax.nn.sigmoid(gi[:, H:2 * H] + gh[:, H:2 * H])
    n_ = jnp.tanh(gi[:, 2 * H:3 * H] + r * gh[:, 2 * H:3 * H])
    hn = (1.0 - z) * n_ + z * h
    h_scr[pl.ds(nb * B, B), :] = hn
    out_ref[...] = hn


def kernel(cat_x, num_x, edges, emb0, emb1, emb2,
           W1, b1, W2, b2, Wih, Whh, bih, bhh):
    f32 = jnp.float32
    i32 = jnp.int32
    cat_t = jnp.transpose(cat_x.astype(i32), (0, 2, 1))
    catp = jnp.pad(cat_t.reshape(T * 3, 125, CH),
                   ((0, 0), (0, 3), (0, 0))).reshape(T * 3 * 128, CH)
    num2 = num_x.reshape(T * N, 32)
    src = edges[:, 0, :].astype(i32)
    dst = edges[:, 1, :].astype(i32)
    epad = E2 - E
    srcp = jnp.concatenate([src, jnp.zeros((T, epad), i32)], axis=1)
    dstp = jnp.concatenate([dst, jnp.full((T, epad), N, i32)], axis=1)
    src_adj = srcp + (jnp.arange(T, dtype=i32) * NP)[:, None]
    srcf = src_adj.reshape(T * E2)
    dstf = dstp.reshape(T * E2)
    dstr = dstp.reshape(T * ER, CH)
    ones80 = jnp.ones((CH, H), f32)
    zeros128 = jnp.zeros((WR, H), f32)
    f0 = jnp.pad(emb0, ((0, 0), (0, 96)))
    f1 = jnp.pad(emb1, ((0, 0), (32, 64)))
    f2 = jnp.pad(emb2, ((0, 0), (64, 32)))

    mesh = plsc.VectorSubcoreMesh(core_axis_name="c", subcore_axis_name="s")

    degp = pl.kernel(
        _deg_body,
        out_type=jax.ShapeDtypeStruct((NC * T * NP, H), f32),
        mesh=mesh,
        scratch_types=[
            pltpu.VMEM((NCH, CH), i32),
            pltpu.VMEM((CH, H), f32),
            pltpu.SemaphoreType.DMA,
            pltpu.VMEM_SHARED((ATOT, H), f32),
        ],
    )(dstr, ones80, zeros128)

    xc012, x3 = pl.kernel(
        _emb_body,
        out_type=(jax.ShapeDtypeStruct((T * NP, H), f32),
                  jax.ShapeDtypeStruct((T * NP, 32), f32)),
        mesh=mesh,
        scratch_types=[
            pltpu.VMEM((8, CH), i32),
            pltpu.VMEM((8, CH), i32),
            pltpu.VMEM((8, CH), i32),
            pltpu.VMEM((CH, H), f32),
            pltpu.VMEM((CH, 32), f32),
            pltpu.SemaphoreType.DMA,
            pltpu.SemaphoreType.DMA,
            pltpu.SemaphoreType.DMA,
            pltpu.SemaphoreType.DMA,
        ],
    )(catp, num2, f0, f1, f2)

    dinvT, y1 = pl.pallas_call(
        _prep_body,
        grid=(T, NB),
        in_specs=[
            pl.BlockSpec((B, H), lambda t, nb: (t * NB + nb, 0)),
            pl.BlockSpec((B, H), lambda t, nb: ((T + t) * NB + nb, 0)),
            pl.BlockSpec((B, H), lambda t, nb: (t * NB + nb, 0)),
            pl.BlockSpec((B, 32), lambda t, nb: (t * NB + nb, 0)),
        ],
        out_specs=[
            pl.BlockSpec((B, 8), lambda t, nb: (t * NB + nb, 0)),
            pl.BlockSpec((B, H), lambda t, nb: (t * NB + nb, 0)),
        ],
        out_shape=[
            jax.ShapeDtypeStruct((T * NP, 8), f32),
            jax.ShapeDtypeStruct((T * NP, H), f32),
        ],
    )(degp, degp, xc012, x3)

    def agg(yarr):
        return pl.kernel(
            _agg_body,
            out_type=jax.ShapeDtypeStruct((NC * T * NP, H), f32),
            mesh=mesh,
            scratch_types=(
                [pltpu.VMEM((CH,), i32) for _ in range(8)]
                + [pltpu.VMEM((CH, H), f32) for _ in range(4)]
                + [pltpu.SemaphoreType.DMA for _ in range(12)]
                + [pltpu.VMEM_SHARED((ATOT, H), f32)]
            ),
        )(yarr, srcf, dstf, zeros128)

    P = agg(y1)

    y2 = pl.pallas_call(
        _mid_body,
        grid=(T, NB),
        in_specs=[
            pl.BlockSpec((B, H), lambda t, nb: (t * NB + nb, 0)),
            pl.BlockSpec((B, H), lambda t, nb: ((T + t) * NB + nb, 0)),
            pl.BlockSpec((B, H), lambda t, nb: (t * NB + nb, 0)),
            pl.BlockSpec((B, 8), lambda t, nb: (t * NB + nb, 0)),
            pl.BlockSpec((1, H, H2), lambda t, nb: (t, 0, 0)),
            pl.BlockSpec((1, 1, H2), lambda t, nb: (t, 0, 0)),
            pl.BlockSpec((1, H2, H), lambda t, nb: (t, 0, 0)),
        ],
        out_specs=pl.BlockSpec((B, H), lambda t, nb: (t * NB + nb, 0)),
        out_shape=jax.ShapeDtypeStruct((T * NP, H), f32),
    )(P, P, y1, dinvT, W1, b1.reshape(T, 1, H2), W2)

    Q = agg(y2)

    h = pl.pallas_call(
        _gru_body,
        grid=(T, NB),
        in_specs=[
            pl.BlockSpec((B, H), lambda t, nb: (t * NB + nb, 0)),
            pl.BlockSpec((B, H), lambda t, nb: ((T + t) * NB + nb, 0)),
            pl.BlockSpec((B, H), lambda t, nb: (t * NB + nb, 0)),
            pl.BlockSpec((B, 8), lambda t, nb: (t * NB + nb, 0)),
            pl.BlockSpec((1, 1, H), lambda t, nb: (t, 0, 0)),
            pl.BlockSpec((H, 3 * H), lambda t, nb: (0, 0)),
            pl.BlockSpec((H, 3 * H), lambda t, nb: (0, 0)),
            pl.BlockSpec((1, 3 * H), lambda t, nb: (0, 0)),
            pl.BlockSpec((1, 3 * H), lambda t, nb: (0, 0)),
        ],
        out_specs=pl.BlockSpec((B, H), lambda t, nb: (nb, 0)),
        out_shape=jax.ShapeDtypeStruct((NP, H), f32),
        scratch_shapes=[pltpu.VMEM((NP, H), f32)],
    )(Q, Q, y2, dinvT, b2.reshape(T, 1, H), jnp.transpose(Wih),
      jnp.transpose(Whh), bih.reshape(1, 3 * H), bhh.reshape(1, 3 * H))
    return h[:N]
